# Initial kernel scaffold; baseline (speedup 1.0000x reference)
import jax
import jax.numpy as jnp
from jax.experimental import pallas as pl


def kernel(a, p, n, table):
    raise NotImplementedError("write your pallas kernel here")


# speedup vs baseline: 1.1388x; 1.1388x over previous
import functools

import jax
import jax.numpy as jnp
from jax import lax
from jax.experimental import pallas as pl
from jax.experimental.pallas import tpu as pltpu
from jax.experimental.pallas import tpu_sc as plsc

_B = 16384
_D = 128
_L = 16
_NC, _NS = 2, 16
_NW = _NC * _NS
_BPW = _B // _NW
_C = 128
_NCHUNK = _BPW // _C

_EPS = 1e-6
_MARGIN = 1.0


def _sc_distances(a, p, n, table):
    mesh = plsc.VectorSubcoreMesh(core_axis_name="c", subcore_axis_name="s")

    @functools.partial(
        pl.kernel,
        mesh=mesh,
        out_type=[
            jax.ShapeDtypeStruct((_B, _L), jnp.float32),
            jax.ShapeDtypeStruct((_B, _L), jnp.float32),
        ],
        scratch_types=[
            pltpu.VMEM((_C,), jnp.int32),
            pltpu.VMEM((_C,), jnp.int32),
            pltpu.VMEM((_C,), jnp.int32),
            pltpu.VMEM((_C, _D), jnp.float32),
            pltpu.VMEM((_C, _D), jnp.float32),
            pltpu.VMEM((_C, _D), jnp.float32),
            pltpu.VMEM((_C, _L), jnp.float32),
            pltpu.VMEM((_C, _L), jnp.float32),
            pltpu.SemaphoreType.DMA,
        ],
    )
    def body(a_h, p_h, n_h, tab_h, outp_h, outn_h,
             ia, ip, inn, ea, ep, en, dp, dn, sem):
        wid = lax.axis_index("s") * _NC + lax.axis_index("c")
        base = wid * _BPW

        def chunk_body(c, carry):
            r0 = base + c * _C
            pltpu.sync_copy(a_h.at[pl.ds(r0, _C)], ia)
            pltpu.sync_copy(p_h.at[pl.ds(r0, _C)], ip)
            pltpu.sync_copy(n_h.at[pl.ds(r0, _C)], inn)
            h1 = pltpu.async_copy(tab_h.at[ia], ea, sem)
            h2 = pltpu.async_copy(tab_h.at[ip], ep, sem)
            h3 = pltpu.async_copy(tab_h.at[inn], en, sem)
            h1.wait()
            h2.wait()
            h3.wait()

            def row_body(r, rcarry):
                accp = jnp.zeros((_L,), jnp.float32)
                accn = jnp.zeros((_L,), jnp.float32)
                for j in range(_D // _L):
                    s = pl.ds(j * _L, _L)
                    va = ea[r, s] + _EPS
                    tp = va - ep[r, s]
                    accp = accp + tp * tp
                    tn = va - en[r, s]
                    accn = accn + tn * tn
                dp[r, :] = accp
                dn[r, :] = accn
                return rcarry

            lax.fori_loop(0, _C, row_body, 0)
            pltpu.sync_copy(dp, outp_h.at[pl.ds(r0, _C)])
            pltpu.sync_copy(dn, outn_h.at[pl.ds(r0, _C)])
            return carry

        lax.fori_loop(0, _NCHUNK, chunk_body, 0)

    return body(a, p, n, table)


def _tc_finish(d2p, d2n):
    def body(dp_ref, dn_ref, o_ref):
        dpos = jnp.sqrt(jnp.sum(dp_ref[...], axis=1))
        dneg = jnp.sqrt(jnp.sum(dn_ref[...], axis=1))
        o_ref[0, 0] = jnp.mean(jnp.maximum(dpos - dneg + _MARGIN, 0.0))

    out = pl.pallas_call(
        body,
        out_shape=jax.ShapeDtypeStruct((1, 1), jnp.float32),
        out_specs=pl.BlockSpec(memory_space=pltpu.SMEM),
    )(d2p, d2n)
    return out[0, 0]


def kernel(a, p, n, table):
    a = a.astype(jnp.int32)
    p = p.astype(jnp.int32)
    n = n.astype(jnp.int32)
    d2p, d2n = _sc_distances(a, p, n, table)
    return _tc_finish(d2p, d2n)
